# Initial kernel scaffold; baseline (speedup 1.0000x reference)
import jax
import jax.numpy as jnp
from jax.experimental import pallas as pl


def kernel(x, w_gate, w_noise, expert_W, expert_b):
    raise NotImplementedError("write your pallas kernel here")


# speedup vs baseline: 1.2750x; 1.2750x over previous
import jax
import jax.numpy as jnp
from jax.experimental import pallas as pl

NUM_EXPERTS = 8
OUT = 1000
FEAT = 1728
IMG_FLAT = 3 * 384 * 384
CHUNK = 12288
N_CHUNKS = IMG_FLAT // CHUNK
FEAT_CHUNK = CHUNK // 256
POOL_ROWS = CHUNK // 384 // 16


def _gate_pool_kernel(x_ref, w_ref, logits_ref, feat_ref):
    i = pl.program_id(0)
    xb = x_ref[...]
    wb = w_ref[...]
    part = jnp.dot(xb, wb, preferred_element_type=jnp.float32)

    B = xb.shape[0]
    xs = xb.reshape(B, CHUNK // 16, 16).sum(axis=2)
    xs = xs.reshape(B, POOL_ROWS, 16, 24).sum(axis=2)
    feat_ref[...] = xs.reshape(1, B, FEAT_CHUNK) * (1.0 / 256.0)

    @pl.when(i == 0)
    def _():
        logits_ref[...] = part

    @pl.when(i > 0)
    def _():
        logits_ref[...] += part


def _moe_kernel(logits_ref, feat_ref, w_ref, b_ref, y_ref):
    e = pl.program_id(0)
    logits = logits_ref[...]
    B = logits.shape[0]
    m = jnp.max(logits, axis=1, keepdims=True)
    ex = jnp.exp(logits - m)
    v = ex / jnp.sum(ex, axis=1, keepdims=True)
    iota = jax.lax.broadcasted_iota(jnp.int32, (B, NUM_EXPERTS), 1)
    m1 = jnp.max(v, axis=1, keepdims=True)
    i1 = jnp.min(jnp.where(v == m1, iota, NUM_EXPERTS), axis=1, keepdims=True)
    vm = jnp.where(iota == i1, jnp.float32(-jnp.inf), v)
    m2 = jnp.max(vm, axis=1, keepdims=True)
    i2 = jnp.min(jnp.where(vm == m2, iota, NUM_EXPERTS), axis=1, keepdims=True)
    denom = m1 + m2 + 1e-6
    g = jnp.where(i1 == e, m1, jnp.where(i2 == e, m2, 0.0)) / denom

    z = jnp.dot(feat_ref[...], w_ref[0], preferred_element_type=jnp.float32)
    contrib = g * (z + b_ref[0])

    @pl.when(e == 0)
    def _():
        y_ref[...] = contrib

    @pl.when(e > 0)
    def _():
        y_ref[...] += contrib


def kernel(x, w_gate, w_noise, expert_W, expert_b):
    del w_noise
    B = x.shape[0]
    x_flat = x.reshape(B, IMG_FLAT)
    logits, feat = pl.pallas_call(
        _gate_pool_kernel,
        grid=(N_CHUNKS,),
        in_specs=[
            pl.BlockSpec((B, CHUNK), lambda i: (0, i)),
            pl.BlockSpec((CHUNK, NUM_EXPERTS), lambda i: (i, 0)),
        ],
        out_specs=[
            pl.BlockSpec((B, NUM_EXPERTS), lambda i: (0, 0)),
            pl.BlockSpec((1, B, FEAT_CHUNK), lambda i: (i, 0, 0)),
        ],
        out_shape=[
            jax.ShapeDtypeStruct((B, NUM_EXPERTS), jnp.float32),
            jax.ShapeDtypeStruct((N_CHUNKS, B, FEAT_CHUNK), jnp.float32),
        ],
    )(x_flat, w_gate)
    feat = feat.transpose(1, 0, 2).reshape(B, FEAT)

    y = pl.pallas_call(
        _moe_kernel,
        grid=(NUM_EXPERTS,),
        in_specs=[
            pl.BlockSpec((B, NUM_EXPERTS), lambda e: (0, 0)),
            pl.BlockSpec((B, FEAT), lambda e: (0, 0)),
            pl.BlockSpec((1, FEAT, OUT), lambda e: (e, 0, 0)),
            pl.BlockSpec((1, 1, OUT), lambda e: (e, 0, 0)),
        ],
        out_specs=pl.BlockSpec((B, OUT), lambda e: (0, 0)),
        out_shape=jax.ShapeDtypeStruct((B, OUT), jnp.float32),
    )(logits, feat, expert_W, expert_b.reshape(NUM_EXPERTS, 1, OUT))
    return y


# speedup vs baseline: 1.8126x; 1.4216x over previous
import jax
import jax.numpy as jnp
import numpy as np
from jax.experimental import pallas as pl

NUM_EXPERTS = 8
OUT = 1000
FEAT = 1728
IMG_FLAT = 3 * 384 * 384
CHUNK = 12288
N_CHUNKS = IMG_FLAT // CHUNK
FEAT_CHUNK = CHUNK // 256

_ph = np.arange(CHUNK) // 384 // 16
_pw = np.arange(CHUNK) % 384 // 16
_POOL_M = np.zeros((CHUNK, FEAT_CHUNK), np.float32)
_POOL_M[np.arange(CHUNK), _ph * 24 + _pw] = 1.0 / 256.0


def _gate_pool_kernel(x_ref, w_ref, m_ref, logits_ref, feat_ref):
    i = pl.program_id(0)
    xb = x_ref[...]
    wb = w_ref[...]
    part = jnp.dot(xb, wb, preferred_element_type=jnp.float32)

    B = xb.shape[0]
    pooled = jnp.dot(xb, m_ref[...], preferred_element_type=jnp.float32)
    feat_ref[...] = pooled.reshape(1, B, FEAT_CHUNK)

    @pl.when(i == 0)
    def _():
        logits_ref[...] = part

    @pl.when(i > 0)
    def _():
        logits_ref[...] += part


def _moe_kernel(logits_ref, feat_ref, w_ref, b_ref, y_ref):
    e = pl.program_id(0)
    logits = logits_ref[...]
    B = logits.shape[0]
    m = jnp.max(logits, axis=1, keepdims=True)
    ex = jnp.exp(logits - m)
    v = ex / jnp.sum(ex, axis=1, keepdims=True)
    iota = jax.lax.broadcasted_iota(jnp.int32, (B, NUM_EXPERTS), 1)
    m1 = jnp.max(v, axis=1, keepdims=True)
    i1 = jnp.min(jnp.where(v == m1, iota, NUM_EXPERTS), axis=1, keepdims=True)
    vm = jnp.where(iota == i1, jnp.float32(-jnp.inf), v)
    m2 = jnp.max(vm, axis=1, keepdims=True)
    i2 = jnp.min(jnp.where(vm == m2, iota, NUM_EXPERTS), axis=1, keepdims=True)
    denom = m1 + m2 + 1e-6
    g = jnp.where(i1 == e, m1, jnp.where(i2 == e, m2, 0.0)) / denom

    z = jnp.dot(feat_ref[...], w_ref[0], preferred_element_type=jnp.float32)
    contrib = g * (z + b_ref[0])

    @pl.when(e == 0)
    def _():
        y_ref[...] = contrib

    @pl.when(e > 0)
    def _():
        y_ref[...] += contrib


def kernel(x, w_gate, w_noise, expert_W, expert_b):
    del w_noise
    B = x.shape[0]
    x_flat = x.reshape(B, IMG_FLAT)
    logits, feat = pl.pallas_call(
        _gate_pool_kernel,
        grid=(N_CHUNKS,),
        in_specs=[
            pl.BlockSpec((B, CHUNK), lambda i: (0, i)),
            pl.BlockSpec((CHUNK, NUM_EXPERTS), lambda i: (i, 0)),
            pl.BlockSpec((CHUNK, FEAT_CHUNK), lambda i: (0, 0)),
        ],
        out_specs=[
            pl.BlockSpec((B, NUM_EXPERTS), lambda i: (0, 0)),
            pl.BlockSpec((1, B, FEAT_CHUNK), lambda i: (i, 0, 0)),
        ],
        out_shape=[
            jax.ShapeDtypeStruct((B, NUM_EXPERTS), jnp.float32),
            jax.ShapeDtypeStruct((N_CHUNKS, B, FEAT_CHUNK), jnp.float32),
        ],
    )(x_flat, w_gate, jnp.asarray(_POOL_M))
    feat = feat.transpose(1, 0, 2).reshape(B, FEAT)

    y = pl.pallas_call(
        _moe_kernel,
        grid=(NUM_EXPERTS,),
        in_specs=[
            pl.BlockSpec((B, NUM_EXPERTS), lambda e: (0, 0)),
            pl.BlockSpec((B, FEAT), lambda e: (0, 0)),
            pl.BlockSpec((1, FEAT, OUT), lambda e: (e, 0, 0)),
            pl.BlockSpec((1, 1, OUT), lambda e: (e, 0, 0)),
        ],
        out_specs=pl.BlockSpec((B, OUT), lambda e: (0, 0)),
        out_shape=jax.ShapeDtypeStruct((B, OUT), jnp.float32),
    )(logits, feat, expert_W, expert_b.reshape(NUM_EXPERTS, 1, OUT))
    return y


# speedup vs baseline: 4.0764x; 2.2489x over previous
import jax
import jax.numpy as jnp
import numpy as np
from jax.experimental import pallas as pl

NUM_EXPERTS = 8
OUT = 1000
FEAT = 1728
IMG_FLAT = 3 * 384 * 384
CHUNK = 12288
N_CHUNKS = IMG_FLAT // CHUNK
FEAT_CHUNK = CHUNK // 256

_ph = np.arange(CHUNK) // 384 // 16
_pw = np.arange(CHUNK) % 384 // 16
_POOL_M = np.zeros((CHUNK, FEAT_CHUNK), np.float32)
_POOL_M[np.arange(CHUNK), _ph * 24 + _pw] = 1.0 / 256.0


def _gate_pool_kernel(x_ref, w_ref, m_ref, logits_ref, feat_ref):
    i = pl.program_id(0)
    xb = x_ref[...]
    wt = w_ref[...]
    part = jax.lax.dot_general(
        xb, wt, (((1,), (1,)), ((), ())),
        preferred_element_type=jnp.float32)

    B = xb.shape[0]
    pooled = jnp.dot(xb, m_ref[...], preferred_element_type=jnp.float32)
    feat_ref[...] = pooled.reshape(1, B, FEAT_CHUNK)

    @pl.when(i == 0)
    def _():
        logits_ref[...] = part

    @pl.when(i > 0)
    def _():
        logits_ref[...] += part


def _moe_kernel(logits_ref, feat_ref, w_ref, b_ref, y_ref):
    e = pl.program_id(0)
    logits = logits_ref[...]
    B = logits.shape[0]
    m = jnp.max(logits, axis=1, keepdims=True)
    ex = jnp.exp(logits - m)
    v = ex / jnp.sum(ex, axis=1, keepdims=True)
    iota = jax.lax.broadcasted_iota(jnp.int32, (B, NUM_EXPERTS), 1)
    m1 = jnp.max(v, axis=1, keepdims=True)
    i1 = jnp.min(jnp.where(v == m1, iota, NUM_EXPERTS), axis=1, keepdims=True)
    vm = jnp.where(iota == i1, jnp.float32(-jnp.inf), v)
    m2 = jnp.max(vm, axis=1, keepdims=True)
    i2 = jnp.min(jnp.where(vm == m2, iota, NUM_EXPERTS), axis=1, keepdims=True)
    denom = m1 + m2 + 1e-6
    g = jnp.where(i1 == e, m1, jnp.where(i2 == e, m2, 0.0)) / denom

    z = jnp.dot(feat_ref[...], w_ref[0], preferred_element_type=jnp.float32)
    contrib = g * (z + b_ref[0])

    @pl.when(e == 0)
    def _():
        y_ref[...] = contrib

    @pl.when(e > 0)
    def _():
        y_ref[...] += contrib


def kernel(x, w_gate, w_noise, expert_W, expert_b):
    del w_noise
    B = x.shape[0]
    x_flat = x.reshape(B, IMG_FLAT)
    logits, feat = pl.pallas_call(
        _gate_pool_kernel,
        grid=(N_CHUNKS,),
        in_specs=[
            pl.BlockSpec((B, CHUNK), lambda i: (0, i)),
            pl.BlockSpec((NUM_EXPERTS, CHUNK), lambda i: (0, i)),
            pl.BlockSpec((CHUNK, FEAT_CHUNK), lambda i: (0, 0)),
        ],
        out_specs=[
            pl.BlockSpec((B, NUM_EXPERTS), lambda i: (0, 0)),
            pl.BlockSpec((1, B, FEAT_CHUNK), lambda i: (i, 0, 0)),
        ],
        out_shape=[
            jax.ShapeDtypeStruct((B, NUM_EXPERTS), jnp.float32),
            jax.ShapeDtypeStruct((N_CHUNKS, B, FEAT_CHUNK), jnp.float32),
        ],
    )(x_flat, w_gate.T, jnp.asarray(_POOL_M))
    feat = feat.transpose(1, 0, 2).reshape(B, FEAT)

    y = pl.pallas_call(
        _moe_kernel,
        grid=(NUM_EXPERTS,),
        in_specs=[
            pl.BlockSpec((B, NUM_EXPERTS), lambda e: (0, 0)),
            pl.BlockSpec((B, FEAT), lambda e: (0, 0)),
            pl.BlockSpec((1, FEAT, OUT), lambda e: (e, 0, 0)),
            pl.BlockSpec((1, 1, OUT), lambda e: (e, 0, 0)),
        ],
        out_specs=pl.BlockSpec((B, OUT), lambda e: (0, 0)),
        out_shape=jax.ShapeDtypeStruct((B, OUT), jnp.float32),
    )(logits, feat, expert_W, expert_b.reshape(NUM_EXPERTS, 1, OUT))
    return y


# speedup vs baseline: 4.6313x; 1.1361x over previous
import jax
import jax.numpy as jnp
import numpy as np
from jax.experimental import pallas as pl
from jax.experimental.pallas import tpu as pltpu

NUM_EXPERTS = 8
OUT = 1000
FEAT = 1728
IMG_FLAT = 3 * 384 * 384
CHUNK = 24576
N_CHUNKS = IMG_FLAT // CHUNK
FEAT_CHUNK = CHUNK // 256

_ph = np.arange(CHUNK) // 384 // 16
_pw = np.arange(CHUNK) % 384 // 16
_POOL_M = np.zeros((CHUNK, FEAT_CHUNK), np.float32)
_POOL_M[np.arange(CHUNK), _ph * 24 + _pw] = 1.0 / 256.0
_POOL_M = _POOL_M.astype(jnp.bfloat16)


def _moe_kernel(x_ref, w_ref, m_ref, we_ref, be_ref, y_ref,
                logits_ref, feat_ref):
    i = pl.program_id(0)

    @pl.when(i < N_CHUNKS)
    def _gate_pool_phase():
        xb = x_ref[...]
        B = xb.shape[0]
        part = jax.lax.dot_general(
            xb, w_ref[...], (((1,), (1,)), ((), ())),
            preferred_element_type=jnp.float32)
        pooled = jax.lax.dot_general(
            xb.astype(jnp.bfloat16), m_ref[...], (((1,), (0,)), ((), ())),
            preferred_element_type=jnp.float32)
        feat_ref[i] = pooled

        @pl.when(i == 0)
        def _():
            logits_ref[...] = part

        @pl.when(i > 0)
        def _():
            logits_ref[...] += part

    @pl.when(i >= N_CHUNKS)
    def _expert_phase():
        e = i - N_CHUNKS
        logits = logits_ref[...]
        B = logits.shape[0]
        m = jnp.max(logits, axis=1, keepdims=True)
        ex = jnp.exp(logits - m)
        v = ex / jnp.sum(ex, axis=1, keepdims=True)
        iota = jax.lax.broadcasted_iota(jnp.int32, (B, NUM_EXPERTS), 1)
        m1 = jnp.max(v, axis=1, keepdims=True)
        i1 = jnp.min(jnp.where(v == m1, iota, NUM_EXPERTS),
                     axis=1, keepdims=True)
        vm = jnp.where(iota == i1, jnp.float32(-jnp.inf), v)
        m2 = jnp.max(vm, axis=1, keepdims=True)
        i2 = jnp.min(jnp.where(vm == m2, iota, NUM_EXPERTS),
                     axis=1, keepdims=True)
        denom = m1 + m2 + 1e-6
        g = jnp.where(i1 == e, m1, jnp.where(i2 == e, m2, 0.0)) / denom

        z = be_ref[0] * jnp.float32(1.0)
        for j in range(N_CHUNKS):
            z = z + jnp.dot(
                feat_ref[j], we_ref[0, j * FEAT_CHUNK:(j + 1) * FEAT_CHUNK],
                preferred_element_type=jnp.float32)
        contrib = g * z

        @pl.when(e == 0)
        def _():
            y_ref[...] = contrib

        @pl.when(e > 0)
        def _():
            y_ref[...] += contrib


def kernel(x, w_gate, w_noise, expert_W, expert_b):
    del w_noise
    B = x.shape[0]
    x_flat = x.reshape(B, IMG_FLAT)
    n_steps = N_CHUNKS + NUM_EXPERTS

    y = pl.pallas_call(
        _moe_kernel,
        grid=(n_steps,),
        in_specs=[
            pl.BlockSpec((B, CHUNK),
                         lambda i: (0, jnp.minimum(i, N_CHUNKS - 1))),
            pl.BlockSpec((NUM_EXPERTS, CHUNK),
                         lambda i: (0, jnp.minimum(i, N_CHUNKS - 1))),
            pl.BlockSpec((CHUNK, FEAT_CHUNK), lambda i: (0, 0)),
            pl.BlockSpec((1, FEAT, OUT),
                         lambda i: (jnp.maximum(i - N_CHUNKS, 0), 0, 0)),
            pl.BlockSpec((1, 1, OUT),
                         lambda i: (jnp.maximum(i - N_CHUNKS, 0), 0, 0)),
        ],
        out_specs=pl.BlockSpec((B, OUT), lambda i: (0, 0)),
        out_shape=jax.ShapeDtypeStruct((B, OUT), jnp.float32),
        scratch_shapes=[
            pltpu.VMEM((B, NUM_EXPERTS), jnp.float32),
            pltpu.VMEM((N_CHUNKS, B, FEAT_CHUNK), jnp.float32),
        ],
    )(x_flat, w_gate.T, jnp.asarray(_POOL_M), expert_W,
      expert_b.reshape(NUM_EXPERTS, 1, OUT))
    return y
